# baseline (speedup 1.0000x reference)
import functools

import jax
import jax.numpy as jnp
from jax import lax
from jax.experimental import pallas as pl
from jax.experimental.pallas import tpu as pltpu
from jax.experimental.pallas import tpu_sc as plsc

SEQ = 2048
UNITS = 32
ROW_W = SEQ * UNITS
NCHUNK = 4
CHUNK_ROWS = SEQ // NCHUNK
CHUNK_W = CHUNK_ROWS * UNITS

_info = plsc.get_sparse_core_info()
_NC = _info.num_cores
_NS = _info.num_subcores
_NW = _NC * _NS
_RPW = SEQ // _NW

_mesh = plsc.VectorSubcoreMesh(core_axis_name="c", subcore_axis_name="s")


@functools.partial(
    pl.kernel,
    mesh=_mesh,
    out_type=jax.ShapeDtypeStruct((SEQ * ROW_W,), jnp.float32),
    scratch_types=[
        pltpu.VMEM((5 * UNITS,), jnp.float32),
        pltpu.VMEM((ROW_W,), jnp.float32),
        pltpu.SemaphoreType.DMA,
        pltpu.SemaphoreType.DMA,
        pltpu.SemaphoreType.DMA,
        pltpu.SemaphoreType.DMA,
    ],
)
def _rel_pos_sc(emb_hbm, out_hbm, emb_v, slab_v, sem0, sem1, sem2, sem3):
    sems = (sem0, sem1, sem2, sem3)
    wid = lax.axis_index("s") * _NC + lax.axis_index("c")
    i0 = wid * _RPW

    pltpu.sync_copy(emb_hbm, emb_v)

    e0a = emb_v[pl.ds(0 * UNITS, 16)]
    e0b = emb_v[pl.ds(0 * UNITS + 16, 16)]
    e1a = emb_v[pl.ds(1 * UNITS, 16)]
    e1b = emb_v[pl.ds(1 * UNITS + 16, 16)]
    e2a = emb_v[pl.ds(2 * UNITS, 16)]
    e2b = emb_v[pl.ds(2 * UNITS + 16, 16)]
    e3a = emb_v[pl.ds(3 * UNITS, 16)]
    e3b = emb_v[pl.ds(3 * UNITS + 16, 16)]
    e4a = emb_v[pl.ds(4 * UNITS, 16)]
    e4b = emb_v[pl.ds(4 * UNITS + 16, 16)]

    def set_row(r, a, b):
        slab_v[pl.ds(r * UNITS, 16)] = a
        slab_v[pl.ds(r * UNITS + 16, 16)] = b

    def fill0(r, _):
        set_row(r, e0a, e0b)
        return 0

    def fill4(r, _):
        set_row(r, e4a, e4b)
        return 0

    lax.fori_loop(0, jnp.maximum(i0 - 1, 0), fill0, 0)

    @pl.when(i0 - 1 >= 0)
    def _():
        set_row(i0 - 1, e1a, e1b)

    set_row(i0, e2a, e2b)

    @pl.when(i0 + 1 < SEQ)
    def _():
        set_row(i0 + 1, e3a, e3b)

    lax.fori_loop(i0 + 2, SEQ, fill4, 0)

    c_lo = jnp.maximum(i0 - 1, 0) // CHUNK_ROWS
    c_hi = jnp.minimum(i0 + _RPW, SEQ - 1) // CHUNK_ROWS

    def drain_one(sem):
        pltpu.make_async_copy(
            out_hbm.at[pl.ds(0, CHUNK_W)],
            slab_v.at[pl.ds(0, CHUNK_W)],
            sem,
        ).wait()

    def body(n, _):
        i = i0 + n

        @pl.when(n > 0)
        def _():
            for c in range(NCHUNK):
                @pl.when((c == c_lo) | (c == c_hi))
                def _(c=c):
                    drain_one(sems[c])

            @pl.when(i - 2 >= 0)
            def _():
                set_row(i - 2, e0a, e0b)

            set_row(i - 1, e1a, e1b)
            set_row(i, e2a, e2b)

            @pl.when(i + 1 < SEQ)
            def _():
                set_row(i + 1, e3a, e3b)

        for c in range(NCHUNK):
            pltpu.async_copy(
                slab_v.at[pl.ds(c * CHUNK_W, CHUNK_W)],
                out_hbm.at[pl.ds(i * ROW_W + c * CHUNK_W, CHUNK_W)],
                sems[c],
            )

        @pl.when(n >= 2)
        def _():
            for c in range(NCHUNK):
                @pl.when((c != c_lo) & (c != c_hi))
                def _(c=c):
                    drain_one(sems[c])

        return 0

    lax.fori_loop(0, _RPW, body, 0)

    for c in range(NCHUNK):
        is_band = (c == c_lo) | (c == c_hi)
        q = jnp.where(is_band, 1, 2)

        def fin(_, __, c=c):
            drain_one(sems[c])
            return 0

        lax.fori_loop(0, q, fin, 0)


def kernel(embedding):
    out = _rel_pos_sc(embedding.reshape(5 * UNITS))
    return out.reshape(SEQ, SEQ, UNITS)


# speedup vs baseline: 1.9766x; 1.9766x over previous
import functools

import jax
import jax.numpy as jnp
from jax import lax
from jax.experimental import pallas as pl
from jax.experimental.pallas import tpu as pltpu
from jax.experimental.pallas import tpu_sc as plsc

SEQ = 2048
UNITS = 32
ROW_W = SEQ * UNITS

_info = plsc.get_sparse_core_info()
_NC = _info.num_cores
_NS = _info.num_subcores
_NW = _NC * _NS
_RPW = SEQ // _NW

_mesh = plsc.VectorSubcoreMesh(core_axis_name="c", subcore_axis_name="s")


@functools.partial(
    pl.kernel,
    mesh=_mesh,
    out_type=jax.ShapeDtypeStruct((SEQ, ROW_W), jnp.float32),
    scratch_types=[
        pltpu.VMEM((5 * UNITS,), jnp.float32),
        pltpu.VMEM((ROW_W,), jnp.float32),
    ],
)
def _rel_pos_sc(emb_hbm, out_hbm, emb_v, slab_v):
    wid = lax.axis_index("s") * _NC + lax.axis_index("c")
    i0 = wid * _RPW

    pltpu.sync_copy(emb_hbm, emb_v)

    e0a = emb_v[pl.ds(0 * UNITS, 16)]
    e0b = emb_v[pl.ds(0 * UNITS + 16, 16)]
    e1a = emb_v[pl.ds(1 * UNITS, 16)]
    e1b = emb_v[pl.ds(1 * UNITS + 16, 16)]
    e2a = emb_v[pl.ds(2 * UNITS, 16)]
    e2b = emb_v[pl.ds(2 * UNITS + 16, 16)]
    e3a = emb_v[pl.ds(3 * UNITS, 16)]
    e3b = emb_v[pl.ds(3 * UNITS + 16, 16)]
    e4a = emb_v[pl.ds(4 * UNITS, 16)]
    e4b = emb_v[pl.ds(4 * UNITS + 16, 16)]

    def set_row(r, a, b):
        slab_v[pl.ds(r * UNITS, 16)] = a
        slab_v[pl.ds(r * UNITS + 16, 16)] = b

    def fill0(r, _):
        set_row(r, e0a, e0b)
        return 0

    def fill4(r, _):
        set_row(r, e4a, e4b)
        return 0

    lax.fori_loop(0, jnp.maximum(i0 - 1, 0), fill0, 0)

    @pl.when(i0 - 1 >= 0)
    def _():
        set_row(i0 - 1, e1a, e1b)

    set_row(i0, e2a, e2b)

    @pl.when(i0 + 1 < SEQ)
    def _():
        set_row(i0 + 1, e3a, e3b)

    lax.fori_loop(i0 + 2, SEQ, fill4, 0)

    def body(n, _):
        i = i0 + n

        @pl.when(n > 0)
        def _():
            @pl.when(i - 2 >= 0)
            def _():
                set_row(i - 2, e0a, e0b)

            set_row(i - 1, e1a, e1b)
            set_row(i, e2a, e2b)

            @pl.when(i + 1 < SEQ)
            def _():
                set_row(i + 1, e3a, e3b)

        pltpu.sync_copy(slab_v, out_hbm.at[i])
        return 0

    lax.fori_loop(0, _RPW, body, 0)


def kernel(embedding):
    out = _rel_pos_sc(embedding.reshape(5 * UNITS))
    return out.reshape(SEQ, SEQ, UNITS)


# speedup vs baseline: 9.9904x; 5.0543x over previous
import functools

import jax
import jax.numpy as jnp
from jax import lax
from jax.experimental import pallas as pl
from jax.experimental.pallas import tpu as pltpu
from jax.experimental.pallas import tpu_sc as plsc

SEQ = 2048
UNITS = 32
WIN = 32

_info = plsc.get_sparse_core_info()
_NC = _info.num_cores
_NS = _info.num_subcores
_NW = _NC * _NS
_RPW = SEQ // _NW

_mesh = plsc.VectorSubcoreMesh(core_axis_name="c", subcore_axis_name="s")


def _win_start(i):
    return jnp.clip(((i - 2) // 16) * 16, 0, SEQ - WIN)


@functools.partial(
    pl.kernel,
    mesh=_mesh,
    out_type=jax.ShapeDtypeStruct((SEQ, UNITS, SEQ), jnp.float32),
    scratch_types=[
        pltpu.VMEM((5 * UNITS,), jnp.float32),
        pltpu.VMEM((UNITS, SEQ), jnp.float32),
    ],
)
def _rel_pos_sc(emb_hbm, out_hbm, emb_v, slab_v):
    wid = lax.axis_index("s") * _NC + lax.axis_index("c")
    i0 = wid * _RPW

    pltpu.sync_copy(emb_hbm, emb_v)

    iota = lax.iota(jnp.int32, 16)

    a0 = _win_start(i0)
    k0_chunks = a0 // 16
    k4_start = a0 // 16 + WIN // 16
    e0 = (emb_v[pl.ds(0, 16)], emb_v[pl.ds(16, 16)])
    e4 = (emb_v[pl.ds(4 * UNITS, 16)], emb_v[pl.ds(4 * UNITS + 16, 16)])
    for u in range(UNITS):
        v0 = jnp.full((16,), e0[u // 16][u % 16], jnp.float32)
        v4 = jnp.full((16,), e4[u // 16][u % 16], jnp.float32)

        def fill0(c, _, u=u, v0=v0):
            slab_v[u, pl.ds(pl.multiple_of(c * 16, 16), 16)] = v0
            return 0

        def fill4(c, _, u=u, v4=v4):
            slab_v[u, pl.ds(pl.multiple_of(c * 16, 16), 16)] = v4
            return 0

        lax.fori_loop(0, k0_chunks, fill0, 0)
        lax.fori_loop(k4_start, SEQ // 16, fill4, 0)

    ev = tuple(
        (emb_v[pl.ds(k * UNITS, 16)], emb_v[pl.ds(k * UNITS + 16, 16)])
        for k in range(5)
    )

    def body(n, _):
        i = i0 + n
        a = _win_start(i)
        masks = []
        for c in range(WIN // 16):
            jv = a + c * 16 + iota
            masks.append((jv <= i - 2, jv == i - 1, jv == i, jv == i + 1))
        for u in range(UNITS):
            v = tuple(
                jnp.full((16,), ev[k][u // 16][u % 16], jnp.float32)
                for k in range(5)
            )
            for c in range(WIN // 16):
                m0, m1, m2, m3 = masks[c]
                vals = jnp.where(
                    m0,
                    v[0],
                    jnp.where(m1, v[1], jnp.where(m2, v[2], jnp.where(m3, v[3], v[4]))),
                )
                slab_v[u, pl.ds(pl.multiple_of(a + c * 16, 16), 16)] = vals
        pltpu.sync_copy(slab_v, out_hbm.at[i])
        return 0

    lax.fori_loop(0, _RPW, body, 0)


def kernel(embedding):
    out = _rel_pos_sc(embedding.reshape(5 * UNITS))
    return jnp.swapaxes(out, 1, 2)
